# Initial kernel scaffold; baseline (speedup 1.0000x reference)
import jax
import jax.numpy as jnp
from jax.experimental import pallas as pl


def kernel(Q, K, V, mask, W_conv):
    raise NotImplementedError("write your pallas kernel here")


# speedup vs baseline: 1.2876x; 1.2876x over previous
import functools
import math

import jax
import jax.numpy as jnp
from jax.experimental import pallas as pl

_HASH_CODE_LEN = 8
_CONV_WINDOW = 5
_EPS = 1e-12

_ACOS_COEFFS = tuple(
    c / math.pi
    for c in (1.5707963050, -0.2145988016, 0.0889789874, -0.0501743046,
              0.0308918810, -0.0170881256, 0.0066700901, -0.0012624911)
)


def _collision_prob(x):
    a = jnp.abs(x)
    poly = _ACOS_COEFFS[-1]
    for c in reversed(_ACOS_COEFFS[:-1]):
        poly = poly * a + c
    v = jnp.sqrt(1.0 - a) * poly
    return jnp.where(x >= 0, 1.0 - v, v)


def _yoso_tile(q_ref, k_ref, vp_ref, m_ref, w_ref, o_ref, *, ts, seq, pad):
    i = pl.program_id(1)
    q = q_ref[0]
    k = k_ref[0]
    vm = vp_ref[0, pad:pad + seq, :]
    mrow = m_ref[0, pl.ds(i * ts, ts)]

    qn = q / jnp.maximum(jnp.sqrt(jnp.sum(q * q, axis=-1, keepdims=True)), _EPS)
    kn = k / jnp.maximum(jnp.sqrt(jnp.sum(k * k, axis=-1, keepdims=True)), _EPS)

    qk = jax.lax.dot_general(qn, kn, (((1,), (1,)), ((), ())),
                             preferred_element_type=jnp.float32)
    qk = jnp.clip(qk, -1.0 + 1e-6, 1.0 - 1e-6)
    p = _collision_prob(qk)
    p2 = p * p
    p4 = p2 * p2
    p8 = p4 * p4

    x = jax.lax.dot_general(p8, vm, (((1,), (0,)), ((), ())),
                            preferred_element_type=jnp.float32)
    x = x * mrow[:, None]
    x = x / jnp.maximum(jnp.sqrt(jnp.sum(x * x, axis=-1, keepdims=True)), _EPS)

    w = w_ref[0, 0]
    conv = jnp.zeros_like(x)
    for j in range(_CONV_WINDOW):
        tap = vp_ref[0, pl.ds(i * ts + j + pad - _CONV_WINDOW // 2, ts), :]
        conv = conv + tap * w[j]

    o_ref[0] = x + conv


def kernel(Q, K, V, mask, W_conv):
    B, H, S, D = Q.shape
    BH = B * H
    TS = 256
    PAD = 4

    Qf = Q.reshape(BH, S, D)
    Kf = K.reshape(BH, S, D)
    Vm = (V * mask[:, None, :, None]).reshape(BH, S, D)
    Vp = jnp.pad(Vm, ((0, 0), (PAD, PAD), (0, 0)))
    Wc = W_conv.reshape(H, 1, _CONV_WINDOW)

    grid = (BH, S // TS)
    out = pl.pallas_call(
        functools.partial(_yoso_tile, ts=TS, seq=S, pad=PAD),
        grid=grid,
        in_specs=[
            pl.BlockSpec((1, TS, D), lambda bh, i: (bh, i, 0)),
            pl.BlockSpec((1, S, D), lambda bh, i: (bh, 0, 0)),
            pl.BlockSpec((1, S + 2 * PAD, D), lambda bh, i: (bh, 0, 0)),
            pl.BlockSpec((1, S), lambda bh, i: (bh // H, 0)),
            pl.BlockSpec((1, 1, _CONV_WINDOW), lambda bh, i: (bh % H, 0, 0)),
        ],
        out_specs=pl.BlockSpec((1, TS, D), lambda bh, i: (bh, i, 0)),
        out_shape=jax.ShapeDtypeStruct((BH, S, D), jnp.float32),
    )(Qf, Kf, Vp, mask, Wc)
    return out.reshape(B, H, S, D)
